# baseline (speedup 1.0000x reference)
import functools

import jax
import jax.numpy as jnp
from jax import lax
from jax.experimental import pallas as pl
from jax.experimental.pallas import tpu as pltpu
from jax.experimental.pallas import tpu_sc as plsc

EMBED = 32
CHUNK = 128
TILE = 128


@functools.lru_cache(maxsize=None)
def _make_detile(N, D):
    info = plsc.get_sparse_core_info()
    NC, NS = info.num_cores, info.num_subcores
    NW = NC * NS
    n_full = N // TILE
    tail = N - n_full * TILE
    base_cnt = n_full // NW
    extra = n_full - base_cnt * NW
    mesh = plsc.VectorSubcoreMesh(core_axis_name="c", subcore_axis_name="s")

    @functools.partial(
        pl.kernel,
        mesh=mesh,
        out_type=[
            jax.ShapeDtypeStruct((N * D,), jnp.float32),
            jax.ShapeDtypeStruct((N * D,), jnp.float32),
        ],
        scratch_types=[
            pltpu.VMEM((D, TILE), jnp.float32),
            pltpu.VMEM((TILE * D,), jnp.float32),
            pltpu.SemaphoreType.DMA,
        ],
        compiler_params=pltpu.CompilerParams(needs_layout_passes=False),
    )
    def detile_k(utabT, itabT, utail, itail, uout, iout, stage, obuf, sem):
        L = 16
        wid = lax.axis_index("s") * NC + lax.axis_index("c")
        count = base_cnt + jnp.minimum(jnp.maximum(extra - wid, 0), 1)
        f_lo = lax.iota(jnp.int32, L)
        f_hi = f_lo + L

        for tabT, tl, out in ((utabT, utail, uout), (itabT, itail, iout)):
            def per_tile(k, _, tabT=tabT, out=out):
                tc = wid + k * NW
                pltpu.sync_copy(tabT.at[:, pl.ds(tc * TILE, TILE)], stage)

                def per_col(c, _):
                    cc = jnp.full((L,), c, jnp.int32)
                    v0 = plsc.load_gather(stage, [f_lo, cc])
                    v1 = plsc.load_gather(stage, [f_hi, cc])
                    obuf[pl.ds(c * D, L)] = v0
                    obuf[pl.ds(c * D + L, L)] = v1
                    return _

                lax.fori_loop(0, TILE, per_col, 0)
                pltpu.sync_copy(obuf, out.at[pl.ds(tc * TILE * D, TILE * D)])
                return _

            lax.fori_loop(0, count, per_tile, 0)
            @pl.when(wid == 0)
            def _copy_tail(tl=tl, out=out):
                pltpu.sync_copy(tl, out.at[pl.ds(n_full * TILE * D, tail * D)])

    return detile_k


@functools.lru_cache(maxsize=None)
def _make_gather(B, D):
    info = plsc.get_sparse_core_info()
    NC, NS = info.num_cores, info.num_subcores
    NW = NC * NS
    b_per_w = B // NW
    n_chunks = b_per_w // CHUNK
    mesh = plsc.VectorSubcoreMesh(core_axis_name="c", subcore_axis_name="s")

    @functools.partial(
        pl.kernel,
        mesh=mesh,
        out_type=[
            jax.ShapeDtypeStruct((B, D), jnp.float32),
            jax.ShapeDtypeStruct((B, D), jnp.float32),
        ],
        scratch_types=[
            pltpu.VMEM((n_chunks, CHUNK), jnp.int32),
            pltpu.VMEM((n_chunks, CHUNK), jnp.int32),
            pltpu.VMEM((b_per_w, D), jnp.float32),
            pltpu.VMEM((b_per_w, D), jnp.float32),
            pltpu.SemaphoreType.DMA,
        ],
        compiler_params=pltpu.CompilerParams(use_tc_tiling_on_sc=False),
    )
    def gather_k(utab, itab, uidx, iidx, uout, iout, uidx_v, iidx_v, urows, irows, sem):
        wid = lax.axis_index("s") * NC + lax.axis_index("c")
        row0 = wid * n_chunks
        pltpu.sync_copy(uidx.at[pl.ds(row0, n_chunks)], uidx_v)
        pltpu.sync_copy(iidx.at[pl.ds(row0, n_chunks)], iidx_v)
        copies = []
        for j in range(n_chunks):
            copies.append(
                pltpu.async_copy(utab.at[uidx_v.at[j]], urows.at[pl.ds(j * CHUNK, CHUNK)], sem)
            )
            copies.append(
                pltpu.async_copy(itab.at[iidx_v.at[j]], irows.at[pl.ds(j * CHUNK, CHUNK)], sem)
            )
        for c in copies:
            c.wait()
        base = wid * b_per_w
        pltpu.sync_copy(urows, uout.at[pl.ds(base, b_per_w)])
        pltpu.sync_copy(irows, iout.at[pl.ds(base, b_per_w)])

    return gather_k


def _desc_body(desc, Wd, bd, out):
    out[...] = jnp.maximum(
        jnp.dot(desc[...], Wd[...], preferred_element_type=jnp.float32) + bd[...], 0.0
    )


def _desc_proj(desc, Wd, bd):
    B, K = desc.shape
    D = Wd.shape[1]
    BB = 2048
    return pl.pallas_call(
        _desc_body,
        grid=(B // BB,),
        in_specs=[
            pl.BlockSpec((BB, K), lambda i: (i, 0)),
            pl.BlockSpec(Wd.shape, lambda i: (0, 0)),
            pl.BlockSpec(bd.shape, lambda i: (0, 0)),
        ],
        out_specs=pl.BlockSpec((BB, D), lambda i: (i, 0)),
        out_shape=jax.ShapeDtypeStruct((B, D), jnp.float32),
    )(desc, Wd, bd)


def _mlp_body(uemb, iemb, dd, W1, b1, W2, b2, Wo, bo, out):
    W1v = W1[...]
    h = (
        jnp.dot(uemb[...], W1v[0:EMBED], preferred_element_type=jnp.float32)
        + jnp.dot(iemb[...], W1v[EMBED : 2 * EMBED], preferred_element_type=jnp.float32)
        + jnp.dot(dd[...], W1v[2 * EMBED :], preferred_element_type=jnp.float32)
        + b1[...]
    )
    h = jnp.maximum(h, 0.0)
    h2 = jnp.maximum(
        jnp.dot(h, W2[...], preferred_element_type=jnp.float32) + b2[...], 0.0
    )
    out[...] = jnp.sum(h2 * Wo[...], axis=1, keepdims=True) + bo[...]


def _mlp(uemb, iemb, dd, W1, b1, W2, b2, Wo, bo):
    B, D = uemb.shape
    BB = 4096

    def row_blk(shape):
        return pl.BlockSpec(shape, lambda i: (i, 0))

    def full_blk(shape):
        return pl.BlockSpec(shape, lambda i: (0, 0))

    return pl.pallas_call(
        _mlp_body,
        grid=(B // BB,),
        in_specs=[
            row_blk((BB, D)),
            row_blk((BB, D)),
            row_blk((BB, D)),
            full_blk(W1.shape),
            full_blk(b1.shape),
            full_blk(W2.shape),
            full_blk(b2.shape),
            full_blk(Wo.shape),
            full_blk(bo.shape),
        ],
        out_specs=row_blk((BB, 1)),
        out_shape=jax.ShapeDtypeStruct((B, 1), jnp.float32),
    )(uemb, iemb, dd, W1, b1, W2, b2, Wo, bo)


@jax.jit
def kernel(user_input, item_input, description_input, user_table, item_table,
           W_desc, b_desc, W1, b1, W2, b2, W_out, b_out):
    B = user_input.shape[0]
    N = user_table.shape[0]
    uidx = user_input.reshape(B // CHUNK, CHUNK)
    iidx = item_input.reshape(B // CHUNK, CHUNK)
    split = (N // TILE) * TILE
    ulin, ilin = _make_detile(N, EMBED)(
        user_table.T, item_table.T,
        user_table[split:].reshape(-1), item_table[split:].reshape(-1),
    )
    uemb, iemb = _make_gather(B, EMBED)(
        ulin.reshape(N, EMBED), ilin.reshape(N, EMBED), uidx, iidx
    )
    dd = _desc_proj(description_input, W_desc, b_desc.reshape(1, -1))
    return _mlp(
        uemb, iemb, dd,
        W1, b1.reshape(1, -1),
        W2, b2.reshape(1, -1),
        W_out.reshape(1, -1), b_out.reshape(1, -1),
    )


# speedup vs baseline: 1.3579x; 1.3579x over previous
import functools

import jax
import jax.numpy as jnp
from jax import lax
from jax.experimental import pallas as pl
from jax.experimental.pallas import tpu as pltpu
from jax.experimental.pallas import tpu_sc as plsc

EMBED = 32
CHUNK = 128
TILE = 128


@functools.lru_cache(maxsize=None)
def _make_detile(N, D):
    info = plsc.get_sparse_core_info()
    NC, NS = info.num_cores, info.num_subcores
    NW = NC * NS
    L = 16
    GRP = 6
    W = GRP * TILE
    n_full = N // TILE
    tail = N - n_full * TILE
    n_sc = n_full // GRP
    assert n_sc * GRP == n_full
    base_cnt = n_sc // NW
    extra = n_sc - base_cnt * NW
    mesh = plsc.VectorSubcoreMesh(core_axis_name="c", subcore_axis_name="s")

    @functools.partial(
        pl.kernel,
        mesh=mesh,
        out_type=[
            jax.ShapeDtypeStruct((N * D,), jnp.float32),
            jax.ShapeDtypeStruct((N * D,), jnp.float32),
        ],
        scratch_types=[
            pltpu.VMEM((D, W), jnp.float32),
            pltpu.VMEM((D, W), jnp.float32),
            pltpu.VMEM((W * D,), jnp.float32),
            pltpu.VMEM((W * D,), jnp.float32),
            pltpu.SemaphoreType.DMA,
            pltpu.SemaphoreType.DMA,
            pltpu.SemaphoreType.DMA,
            pltpu.SemaphoreType.DMA,
        ],
        compiler_params=pltpu.CompilerParams(needs_layout_passes=False),
    )
    def detile_k(utabT, itabT, utail, itail, uout, iout,
                 st0, st1, ob0, ob1, ls0, ls1, ws0, ws1):
        wid = lax.axis_index("s") * NC + lax.axis_index("c")
        count = base_cnt + jnp.minimum(jnp.maximum(extra - wid, 0), 1)
        f_lo = lax.iota(jnp.int32, L)
        f_hi = f_lo + L
        stages = (st0, st1)
        obufs = (ob0, ob1)
        lsems = (ls0, ls1)
        wsems = (ws0, ws1)

        for tabT, tl, out in ((utabT, utail, uout), (itabT, itail, iout)):
            def load(i, slot, tabT=tabT):
                @pl.when(i < count)
                def _():
                    sc = wid + i * NW
                    pltpu.async_copy(
                        tabT.at[:, pl.ds(sc * W, W)], stages[slot], lsems[slot]
                    )

            def proc(i, slot, tabT=tabT, out=out):
                @pl.when(i < count)
                def _():
                    sc = wid + i * NW
                    stage, obuf = stages[slot], obufs[slot]
                    pltpu.make_async_copy(
                        tabT.at[:, pl.ds(0, W)], stage, lsems[slot]
                    ).wait()

                    def per_grp(g, _):
                        cb = g * 8
                        for cj in range(8):
                            cc = jnp.full((L,), cb + cj, jnp.int32)
                            v0 = plsc.load_gather(stage, [f_lo, cc])
                            v1 = plsc.load_gather(stage, [f_hi, cc])
                            obuf[pl.ds((cb + cj) * D, L)] = v0
                            obuf[pl.ds((cb + cj) * D + L, L)] = v1
                        return _

                    lax.fori_loop(0, W // 8, per_grp, 0)
                    @pl.when(i >= 2)
                    def _w():
                        pltpu.make_async_copy(
                            out.at[pl.ds(0, W * D)], obuf, wsems[slot]
                        ).wait()

                    pltpu.async_copy(
                        obuf, out.at[pl.ds(sc * W * D, W * D)], wsems[slot]
                    )

            load(0, 0)

            def pair(g, _):
                i0, i1 = 2 * g, 2 * g + 1
                load(i1, 1)
                proc(i0, 0)
                load(i1 + 1, 0)
                proc(i1, 1)
                return _

            lax.fori_loop(0, (base_cnt + 2) // 2, pair, 0)
            for slot in (0, 1):
                @pl.when(count > slot)
                def _fd(slot=slot, out=out):
                    pltpu.make_async_copy(
                        out.at[pl.ds(0, W * D)], obufs[slot], wsems[slot]
                    ).wait()

            @pl.when(wid == 0)
            def _copy_tail(tl=tl, out=out):
                pltpu.sync_copy(tl, out.at[pl.ds(n_full * TILE * D, tail * D)])

    return detile_k


@functools.lru_cache(maxsize=None)
def _make_gather(B, D):
    info = plsc.get_sparse_core_info()
    NC, NS = info.num_cores, info.num_subcores
    NW = NC * NS
    b_per_w = B // NW
    n_chunks = b_per_w // CHUNK
    mesh = plsc.VectorSubcoreMesh(core_axis_name="c", subcore_axis_name="s")

    @functools.partial(
        pl.kernel,
        mesh=mesh,
        out_type=[
            jax.ShapeDtypeStruct((B, D), jnp.float32),
            jax.ShapeDtypeStruct((B, D), jnp.float32),
        ],
        scratch_types=[
            pltpu.VMEM((n_chunks, CHUNK), jnp.int32),
            pltpu.VMEM((n_chunks, CHUNK), jnp.int32),
            pltpu.VMEM((b_per_w, D), jnp.float32),
            pltpu.VMEM((b_per_w, D), jnp.float32),
            pltpu.SemaphoreType.DMA,
        ],
        compiler_params=pltpu.CompilerParams(use_tc_tiling_on_sc=False),
    )
    def gather_k(utab, itab, uidx, iidx, uout, iout, uidx_v, iidx_v, urows, irows, sem):
        wid = lax.axis_index("s") * NC + lax.axis_index("c")
        row0 = wid * n_chunks
        pltpu.sync_copy(uidx.at[pl.ds(row0, n_chunks)], uidx_v)
        pltpu.sync_copy(iidx.at[pl.ds(row0, n_chunks)], iidx_v)
        copies = []
        for j in range(n_chunks):
            copies.append(
                pltpu.async_copy(utab.at[uidx_v.at[j]], urows.at[pl.ds(j * CHUNK, CHUNK)], sem)
            )
            copies.append(
                pltpu.async_copy(itab.at[iidx_v.at[j]], irows.at[pl.ds(j * CHUNK, CHUNK)], sem)
            )
        for c in copies:
            c.wait()
        base = wid * b_per_w
        pltpu.sync_copy(urows, uout.at[pl.ds(base, b_per_w)])
        pltpu.sync_copy(irows, iout.at[pl.ds(base, b_per_w)])

    return gather_k


def _desc_body(desc, Wd, bd, out):
    out[...] = jnp.maximum(
        jnp.dot(desc[...], Wd[...], preferred_element_type=jnp.float32) + bd[...], 0.0
    )


def _desc_proj(desc, Wd, bd):
    B, K = desc.shape
    D = Wd.shape[1]
    BB = 2048
    return pl.pallas_call(
        _desc_body,
        grid=(B // BB,),
        in_specs=[
            pl.BlockSpec((BB, K), lambda i: (i, 0)),
            pl.BlockSpec(Wd.shape, lambda i: (0, 0)),
            pl.BlockSpec(bd.shape, lambda i: (0, 0)),
        ],
        out_specs=pl.BlockSpec((BB, D), lambda i: (i, 0)),
        out_shape=jax.ShapeDtypeStruct((B, D), jnp.float32),
    )(desc, Wd, bd)


def _mlp_body(uemb, iemb, dd, W1, b1, W2, b2, Wo, bo, out):
    W1v = W1[...]
    h = (
        jnp.dot(uemb[...], W1v[0:EMBED], preferred_element_type=jnp.float32)
        + jnp.dot(iemb[...], W1v[EMBED : 2 * EMBED], preferred_element_type=jnp.float32)
        + jnp.dot(dd[...], W1v[2 * EMBED :], preferred_element_type=jnp.float32)
        + b1[...]
    )
    h = jnp.maximum(h, 0.0)
    h2 = jnp.maximum(
        jnp.dot(h, W2[...], preferred_element_type=jnp.float32) + b2[...], 0.0
    )
    out[...] = jnp.sum(h2 * Wo[...], axis=1, keepdims=True) + bo[...]


def _mlp(uemb, iemb, dd, W1, b1, W2, b2, Wo, bo):
    B, D = uemb.shape
    BB = 4096

    def row_blk(shape):
        return pl.BlockSpec(shape, lambda i: (i, 0))

    def full_blk(shape):
        return pl.BlockSpec(shape, lambda i: (0, 0))

    return pl.pallas_call(
        _mlp_body,
        grid=(B // BB,),
        in_specs=[
            row_blk((BB, D)),
            row_blk((BB, D)),
            row_blk((BB, D)),
            full_blk(W1.shape),
            full_blk(b1.shape),
            full_blk(W2.shape),
            full_blk(b2.shape),
            full_blk(Wo.shape),
            full_blk(bo.shape),
        ],
        out_specs=row_blk((BB, 1)),
        out_shape=jax.ShapeDtypeStruct((B, 1), jnp.float32),
    )(uemb, iemb, dd, W1, b1, W2, b2, Wo, bo)


@jax.jit
def kernel(user_input, item_input, description_input, user_table, item_table,
           W_desc, b_desc, W1, b1, W2, b2, W_out, b_out):
    B = user_input.shape[0]
    N = user_table.shape[0]
    uidx = user_input.reshape(B // CHUNK, CHUNK)
    iidx = item_input.reshape(B // CHUNK, CHUNK)
    split = (N // TILE) * TILE
    ulin, ilin = _make_detile(N, EMBED)(
        user_table.T, item_table.T,
        user_table[split:].reshape(-1), item_table[split:].reshape(-1),
    )
    uemb, iemb = _make_gather(B, EMBED)(
        ulin.reshape(N, EMBED), ilin.reshape(N, EMBED), uidx, iidx
    )
    dd = _desc_proj(description_input, W_desc, b_desc.reshape(1, -1))
    return _mlp(
        uemb, iemb, dd,
        W1, b1.reshape(1, -1),
        W2, b2.reshape(1, -1),
        W_out.reshape(1, -1), b_out.reshape(1, -1),
    )


# speedup vs baseline: 3.3213x; 2.4459x over previous
import functools

import jax
import jax.numpy as jnp
from jax import lax
from jax.experimental import pallas as pl
from jax.experimental.pallas import tpu as pltpu
from jax.experimental.pallas import tpu_sc as plsc

EMBED = 32
CHUNK = 128
TILE = 128


@functools.lru_cache(maxsize=None)
def _make_detile(N, D):
    info = plsc.get_sparse_core_info()
    NC, NS = info.num_cores, info.num_subcores
    NW = NC * NS
    L = 16
    GRP = 6
    W = GRP * TILE
    n_full = N // TILE
    tail = N - n_full * TILE
    n_sc = n_full // GRP
    assert n_sc * GRP == n_full
    base_cnt = n_sc // NW
    extra = n_sc - base_cnt * NW
    mesh = plsc.VectorSubcoreMesh(core_axis_name="c", subcore_axis_name="s")

    @functools.partial(
        pl.kernel,
        mesh=mesh,
        out_type=[
            jax.ShapeDtypeStruct((N * D,), jnp.float32),
            jax.ShapeDtypeStruct((N * D,), jnp.float32),
        ],
        scratch_types=[
            pltpu.VMEM((D, W), jnp.float32),
            pltpu.VMEM((D, W), jnp.float32),
            pltpu.VMEM((W * D,), jnp.float32),
            pltpu.VMEM((W * D,), jnp.float32),
            pltpu.SemaphoreType.DMA,
            pltpu.SemaphoreType.DMA,
            pltpu.SemaphoreType.DMA,
            pltpu.SemaphoreType.DMA,
        ],
        compiler_params=pltpu.CompilerParams(needs_layout_passes=False),
    )
    def detile_k(utabT, itabT, utail, itail, uout, iout,
                 st0, st1, ob0, ob1, ls0, ls1, ws0, ws1):
        wid = lax.axis_index("s") * NC + lax.axis_index("c")
        count = base_cnt + jnp.minimum(jnp.maximum(extra - wid, 0), 1)
        f_lo = lax.iota(jnp.int32, L)
        f_hi = f_lo + L
        stages = (st0, st1)
        obufs = (ob0, ob1)
        lsems = (ls0, ls1)
        wsems = (ws0, ws1)

        for tabT, tl, out in ((utabT, utail, uout), (itabT, itail, iout)):
            def load(i, slot, tabT=tabT):
                @pl.when(i < count)
                def _():
                    sc = wid + i * NW
                    pltpu.async_copy(
                        tabT.at[:, pl.ds(sc * W, W)], stages[slot], lsems[slot]
                    )

            def proc(i, slot, tabT=tabT, out=out):
                @pl.when(i < count)
                def _():
                    sc = wid + i * NW
                    stage, obuf = stages[slot], obufs[slot]
                    pltpu.make_async_copy(
                        tabT.at[:, pl.ds(0, W)], stage, lsems[slot]
                    ).wait()

                    def per_blk(b, _):
                        cvec = b * L + f_lo
                        pos_base = cvec * D
                        for f0 in (0, L):
                            for d in range(L):
                                fvec = f0 + jnp.bitwise_and(d + f_lo, L - 1)
                                v = plsc.load_gather(stage, [fvec, cvec])
                                plsc.store_scatter(obuf, [pos_base + fvec], v)
                        return _

                    lax.fori_loop(0, W // L, per_blk, 0)
                    @pl.when(i >= 2)
                    def _w():
                        pltpu.make_async_copy(
                            out.at[pl.ds(0, W * D)], obuf, wsems[slot]
                        ).wait()

                    pltpu.async_copy(
                        obuf, out.at[pl.ds(sc * W * D, W * D)], wsems[slot]
                    )

            load(0, 0)

            def pair(g, _):
                i0, i1 = 2 * g, 2 * g + 1
                load(i1, 1)
                proc(i0, 0)
                load(i1 + 1, 0)
                proc(i1, 1)
                return _

            lax.fori_loop(0, (base_cnt + 2) // 2, pair, 0)
            for slot in (0, 1):
                @pl.when(count > slot)
                def _fd(slot=slot, out=out):
                    pltpu.make_async_copy(
                        out.at[pl.ds(0, W * D)], obufs[slot], wsems[slot]
                    ).wait()

            @pl.when(wid == 0)
            def _copy_tail(tl=tl, out=out):
                pltpu.sync_copy(tl, out.at[pl.ds(n_full * TILE * D, tail * D)])

    return detile_k


@functools.lru_cache(maxsize=None)
def _make_gather(B, D):
    info = plsc.get_sparse_core_info()
    NC, NS = info.num_cores, info.num_subcores
    NW = NC * NS
    b_per_w = B // NW
    n_chunks = b_per_w // CHUNK
    mesh = plsc.VectorSubcoreMesh(core_axis_name="c", subcore_axis_name="s")

    @functools.partial(
        pl.kernel,
        mesh=mesh,
        out_type=[
            jax.ShapeDtypeStruct((B, D), jnp.float32),
            jax.ShapeDtypeStruct((B, D), jnp.float32),
        ],
        scratch_types=[
            pltpu.VMEM((n_chunks, CHUNK), jnp.int32),
            pltpu.VMEM((n_chunks, CHUNK), jnp.int32),
            pltpu.VMEM((b_per_w, D), jnp.float32),
            pltpu.VMEM((b_per_w, D), jnp.float32),
            pltpu.SemaphoreType.DMA,
        ],
        compiler_params=pltpu.CompilerParams(use_tc_tiling_on_sc=False),
    )
    def gather_k(utab, itab, uidx, iidx, uout, iout, uidx_v, iidx_v, urows, irows, sem):
        wid = lax.axis_index("s") * NC + lax.axis_index("c")
        row0 = wid * n_chunks
        pltpu.sync_copy(uidx.at[pl.ds(row0, n_chunks)], uidx_v)
        pltpu.sync_copy(iidx.at[pl.ds(row0, n_chunks)], iidx_v)
        copies = []
        for j in range(n_chunks):
            copies.append(
                pltpu.async_copy(utab.at[uidx_v.at[j]], urows.at[pl.ds(j * CHUNK, CHUNK)], sem)
            )
            copies.append(
                pltpu.async_copy(itab.at[iidx_v.at[j]], irows.at[pl.ds(j * CHUNK, CHUNK)], sem)
            )
        for c in copies:
            c.wait()
        base = wid * b_per_w
        pltpu.sync_copy(urows, uout.at[pl.ds(base, b_per_w)])
        pltpu.sync_copy(irows, iout.at[pl.ds(base, b_per_w)])

    return gather_k


def _desc_body(desc, Wd, bd, out):
    out[...] = jnp.maximum(
        jnp.dot(desc[...], Wd[...], preferred_element_type=jnp.float32) + bd[...], 0.0
    )


def _desc_proj(desc, Wd, bd):
    B, K = desc.shape
    D = Wd.shape[1]
    BB = 2048
    return pl.pallas_call(
        _desc_body,
        grid=(B // BB,),
        in_specs=[
            pl.BlockSpec((BB, K), lambda i: (i, 0)),
            pl.BlockSpec(Wd.shape, lambda i: (0, 0)),
            pl.BlockSpec(bd.shape, lambda i: (0, 0)),
        ],
        out_specs=pl.BlockSpec((BB, D), lambda i: (i, 0)),
        out_shape=jax.ShapeDtypeStruct((B, D), jnp.float32),
    )(desc, Wd, bd)


def _mlp_body(uemb, iemb, dd, W1, b1, W2, b2, Wo, bo, out):
    W1v = W1[...]
    h = (
        jnp.dot(uemb[...], W1v[0:EMBED], preferred_element_type=jnp.float32)
        + jnp.dot(iemb[...], W1v[EMBED : 2 * EMBED], preferred_element_type=jnp.float32)
        + jnp.dot(dd[...], W1v[2 * EMBED :], preferred_element_type=jnp.float32)
        + b1[...]
    )
    h = jnp.maximum(h, 0.0)
    h2 = jnp.maximum(
        jnp.dot(h, W2[...], preferred_element_type=jnp.float32) + b2[...], 0.0
    )
    out[...] = jnp.sum(h2 * Wo[...], axis=1, keepdims=True) + bo[...]


def _mlp(uemb, iemb, dd, W1, b1, W2, b2, Wo, bo):
    B, D = uemb.shape
    BB = 4096

    def row_blk(shape):
        return pl.BlockSpec(shape, lambda i: (i, 0))

    def full_blk(shape):
        return pl.BlockSpec(shape, lambda i: (0, 0))

    return pl.pallas_call(
        _mlp_body,
        grid=(B // BB,),
        in_specs=[
            row_blk((BB, D)),
            row_blk((BB, D)),
            row_blk((BB, D)),
            full_blk(W1.shape),
            full_blk(b1.shape),
            full_blk(W2.shape),
            full_blk(b2.shape),
            full_blk(Wo.shape),
            full_blk(bo.shape),
        ],
        out_specs=row_blk((BB, 1)),
        out_shape=jax.ShapeDtypeStruct((B, 1), jnp.float32),
    )(uemb, iemb, dd, W1, b1, W2, b2, Wo, bo)


@jax.jit
def kernel(user_input, item_input, description_input, user_table, item_table,
           W_desc, b_desc, W1, b1, W2, b2, W_out, b_out):
    B = user_input.shape[0]
    N = user_table.shape[0]
    uidx = user_input.reshape(B // CHUNK, CHUNK)
    iidx = item_input.reshape(B // CHUNK, CHUNK)
    split = (N // TILE) * TILE
    ulin, ilin = _make_detile(N, EMBED)(
        user_table.T, item_table.T,
        user_table[split:].reshape(-1), item_table[split:].reshape(-1),
    )
    uemb, iemb = _make_gather(B, EMBED)(
        ulin.reshape(N, EMBED), ilin.reshape(N, EMBED), uidx, iidx
    )
    dd = _desc_proj(description_input, W_desc, b_desc.reshape(1, -1))
    return _mlp(
        uemb, iemb, dd,
        W1, b1.reshape(1, -1),
        W2, b2.reshape(1, -1),
        W_out.reshape(1, -1), b_out.reshape(1, -1),
    )


# speedup vs baseline: 7.2238x; 2.1750x over previous
import functools

import jax
import jax.numpy as jnp
from jax import lax
from jax.experimental import pallas as pl
from jax.experimental.pallas import tpu as pltpu
from jax.experimental.pallas import tpu_sc as plsc

EMBED = 32
CHUNK = 128
TILE = 128


@functools.lru_cache(maxsize=None)
def _make_detile(N, D):
    info = plsc.get_sparse_core_info()
    NC, NS = info.num_cores, info.num_subcores
    NW = NC * NS
    L = 16
    GRP = 6
    W = GRP * TILE
    n_full = N // TILE
    tail = N - n_full * TILE
    n_sc = n_full // GRP
    assert n_sc * GRP == n_full
    base_cnt = n_sc // NW
    extra = n_sc - base_cnt * NW
    mesh = plsc.VectorSubcoreMesh(core_axis_name="c", subcore_axis_name="s")

    @functools.partial(
        pl.kernel,
        mesh=mesh,
        out_type=[
            jax.ShapeDtypeStruct((N * D,), jnp.float32),
            jax.ShapeDtypeStruct((N * D,), jnp.float32),
        ],
        scratch_types=[
            pltpu.VMEM((D, W), jnp.float32),
            pltpu.VMEM((D, W), jnp.float32),
            pltpu.VMEM((W * D,), jnp.float32),
            pltpu.VMEM((W * D,), jnp.float32),
            pltpu.SemaphoreType.DMA,
            pltpu.SemaphoreType.DMA,
            pltpu.SemaphoreType.DMA,
            pltpu.SemaphoreType.DMA,
        ],
        compiler_params=pltpu.CompilerParams(needs_layout_passes=False),
    )
    def detile_k(utabT, itabT, utail, itail, uout, iout,
                 st0, st1, ob0, ob1, ls0, ls1, ws0, ws1):
        wid = lax.axis_index("s") * NC + lax.axis_index("c")
        count = base_cnt + jnp.minimum(jnp.maximum(extra - wid, 0), 1)
        f_lo = lax.iota(jnp.int32, L)
        fvecs = [
            f0 + jnp.bitwise_and(d + f_lo, L - 1)
            for f0 in (0, L)
            for d in range(L)
        ]
        stages = (st0, st1)
        obufs = (ob0, ob1)
        lsems = (ls0, ls1)
        wsems = (ws0, ws1)

        for tabT, tl, out in ((utabT, utail, uout), (itabT, itail, iout)):
            def load(i, slot, tabT=tabT):
                @pl.when(i < count)
                def _():
                    sc = wid + i * NW
                    pltpu.async_copy(
                        tabT.at[:, pl.ds(sc * W, W)], stages[slot], lsems[slot]
                    )

            def proc(i, slot, tabT=tabT, out=out):
                @pl.when(i < count)
                def _():
                    sc = wid + i * NW
                    stage, obuf = stages[slot], obufs[slot]
                    pltpu.make_async_copy(
                        tabT.at[:, pl.ds(0, W)], stage, lsems[slot]
                    ).wait()

                    @functools.partial(plsc.parallel_loop, 0, W // L, unroll=2)
                    def _per_blk(b):
                        cvec = b * L + f_lo
                        pos_base = cvec * D
                        for fv in fvecs:
                            v = plsc.load_gather(stage, [fv, cvec])
                            plsc.store_scatter(obuf, [pos_base + fv], v)
                    @pl.when(i >= 2)
                    def _w():
                        pltpu.make_async_copy(
                            out.at[pl.ds(0, W * D)], obuf, wsems[slot]
                        ).wait()

                    pltpu.async_copy(
                        obuf, out.at[pl.ds(sc * W * D, W * D)], wsems[slot]
                    )

            load(0, 0)

            def pair(g, _):
                i0, i1 = 2 * g, 2 * g + 1
                load(i1, 1)
                proc(i0, 0)
                load(i1 + 1, 0)
                proc(i1, 1)
                return _

            lax.fori_loop(0, (base_cnt + 2) // 2, pair, 0)
            for slot in (0, 1):
                @pl.when(count > slot)
                def _fd(slot=slot, out=out):
                    pltpu.make_async_copy(
                        out.at[pl.ds(0, W * D)], obufs[slot], wsems[slot]
                    ).wait()

            @pl.when(wid == 0)
            def _copy_tail(tl=tl, out=out):
                pltpu.sync_copy(tl, out.at[pl.ds(n_full * TILE * D, tail * D)])

    return detile_k


@functools.lru_cache(maxsize=None)
def _make_gather(B, D):
    info = plsc.get_sparse_core_info()
    NC, NS = info.num_cores, info.num_subcores
    NW = NC * NS
    b_per_w = B // NW
    n_chunks = b_per_w // CHUNK
    mesh = plsc.VectorSubcoreMesh(core_axis_name="c", subcore_axis_name="s")

    @functools.partial(
        pl.kernel,
        mesh=mesh,
        out_type=[
            jax.ShapeDtypeStruct((B, D), jnp.float32),
            jax.ShapeDtypeStruct((B, D), jnp.float32),
        ],
        scratch_types=[
            pltpu.VMEM((n_chunks, CHUNK), jnp.int32),
            pltpu.VMEM((n_chunks, CHUNK), jnp.int32),
            pltpu.VMEM((b_per_w, D), jnp.float32),
            pltpu.VMEM((b_per_w, D), jnp.float32),
            pltpu.SemaphoreType.DMA,
        ],
        compiler_params=pltpu.CompilerParams(use_tc_tiling_on_sc=False),
    )
    def gather_k(utab, itab, uidx, iidx, uout, iout, uidx_v, iidx_v, urows, irows, sem):
        wid = lax.axis_index("s") * NC + lax.axis_index("c")
        row0 = wid * n_chunks
        pltpu.sync_copy(uidx.at[pl.ds(row0, n_chunks)], uidx_v)
        pltpu.sync_copy(iidx.at[pl.ds(row0, n_chunks)], iidx_v)
        copies = []
        for j in range(n_chunks):
            copies.append(
                pltpu.async_copy(utab.at[uidx_v.at[j]], urows.at[pl.ds(j * CHUNK, CHUNK)], sem)
            )
            copies.append(
                pltpu.async_copy(itab.at[iidx_v.at[j]], irows.at[pl.ds(j * CHUNK, CHUNK)], sem)
            )
        for c in copies:
            c.wait()
        base = wid * b_per_w
        pltpu.sync_copy(urows, uout.at[pl.ds(base, b_per_w)])
        pltpu.sync_copy(irows, iout.at[pl.ds(base, b_per_w)])

    return gather_k


def _desc_body(desc, Wd, bd, out):
    out[...] = jnp.maximum(
        jnp.dot(desc[...], Wd[...], preferred_element_type=jnp.float32) + bd[...], 0.0
    )


def _desc_proj(desc, Wd, bd):
    B, K = desc.shape
    D = Wd.shape[1]
    BB = 2048
    return pl.pallas_call(
        _desc_body,
        grid=(B // BB,),
        in_specs=[
            pl.BlockSpec((BB, K), lambda i: (i, 0)),
            pl.BlockSpec(Wd.shape, lambda i: (0, 0)),
            pl.BlockSpec(bd.shape, lambda i: (0, 0)),
        ],
        out_specs=pl.BlockSpec((BB, D), lambda i: (i, 0)),
        out_shape=jax.ShapeDtypeStruct((B, D), jnp.float32),
    )(desc, Wd, bd)


def _mlp_body(uemb, iemb, dd, W1, b1, W2, b2, Wo, bo, out):
    W1v = W1[...]
    h = (
        jnp.dot(uemb[...], W1v[0:EMBED], preferred_element_type=jnp.float32)
        + jnp.dot(iemb[...], W1v[EMBED : 2 * EMBED], preferred_element_type=jnp.float32)
        + jnp.dot(dd[...], W1v[2 * EMBED :], preferred_element_type=jnp.float32)
        + b1[...]
    )
    h = jnp.maximum(h, 0.0)
    h2 = jnp.maximum(
        jnp.dot(h, W2[...], preferred_element_type=jnp.float32) + b2[...], 0.0
    )
    out[...] = jnp.sum(h2 * Wo[...], axis=1, keepdims=True) + bo[...]


def _mlp(uemb, iemb, dd, W1, b1, W2, b2, Wo, bo):
    B, D = uemb.shape
    BB = 4096

    def row_blk(shape):
        return pl.BlockSpec(shape, lambda i: (i, 0))

    def full_blk(shape):
        return pl.BlockSpec(shape, lambda i: (0, 0))

    return pl.pallas_call(
        _mlp_body,
        grid=(B // BB,),
        in_specs=[
            row_blk((BB, D)),
            row_blk((BB, D)),
            row_blk((BB, D)),
            full_blk(W1.shape),
            full_blk(b1.shape),
            full_blk(W2.shape),
            full_blk(b2.shape),
            full_blk(Wo.shape),
            full_blk(bo.shape),
        ],
        out_specs=row_blk((BB, 1)),
        out_shape=jax.ShapeDtypeStruct((B, 1), jnp.float32),
    )(uemb, iemb, dd, W1, b1, W2, b2, Wo, bo)


@jax.jit
def kernel(user_input, item_input, description_input, user_table, item_table,
           W_desc, b_desc, W1, b1, W2, b2, W_out, b_out):
    B = user_input.shape[0]
    N = user_table.shape[0]
    uidx = user_input.reshape(B // CHUNK, CHUNK)
    iidx = item_input.reshape(B // CHUNK, CHUNK)
    split = (N // TILE) * TILE
    ulin, ilin = _make_detile(N, EMBED)(
        user_table.T, item_table.T,
        user_table[split:].reshape(-1), item_table[split:].reshape(-1),
    )
    uemb, iemb = _make_gather(B, EMBED)(
        ulin.reshape(N, EMBED), ilin.reshape(N, EMBED), uidx, iidx
    )
    dd = _desc_proj(description_input, W_desc, b_desc.reshape(1, -1))
    return _mlp(
        uemb, iemb, dd,
        W1, b1.reshape(1, -1),
        W2, b2.reshape(1, -1),
        W_out.reshape(1, -1), b_out.reshape(1, -1),
    )


# speedup vs baseline: 7.2250x; 1.0002x over previous
import functools

import jax
import jax.numpy as jnp
from jax import lax
from jax.experimental import pallas as pl
from jax.experimental.pallas import tpu as pltpu
from jax.experimental.pallas import tpu_sc as plsc

EMBED = 32
CHUNK = 128
TILE = 128


@functools.lru_cache(maxsize=None)
def _make_detile(N, D):
    info = plsc.get_sparse_core_info()
    NC, NS = info.num_cores, info.num_subcores
    NW = NC * NS
    L = 16
    GRP = 6
    W = GRP * TILE
    n_full = N // TILE
    tail = N - n_full * TILE
    n_sc = n_full // GRP
    assert n_sc * GRP == n_full
    base_cnt = n_sc // NW
    extra = n_sc - base_cnt * NW
    mesh = plsc.VectorSubcoreMesh(core_axis_name="c", subcore_axis_name="s")

    @functools.partial(
        pl.kernel,
        mesh=mesh,
        out_type=[
            jax.ShapeDtypeStruct((N * D,), jnp.float32),
            jax.ShapeDtypeStruct((N * D,), jnp.float32),
        ],
        scratch_types=[
            pltpu.VMEM((D, W), jnp.float32),
            pltpu.VMEM((D, W), jnp.float32),
            pltpu.VMEM((W * D,), jnp.float32),
            pltpu.VMEM((W * D,), jnp.float32),
            pltpu.SemaphoreType.DMA,
            pltpu.SemaphoreType.DMA,
            pltpu.SemaphoreType.DMA,
            pltpu.SemaphoreType.DMA,
        ],
        compiler_params=pltpu.CompilerParams(needs_layout_passes=False),
    )
    def detile_k(utabT, itabT, utail, itail, uout, iout,
                 st0, st1, ob0, ob1, ls0, ls1, ws0, ws1):
        wid = lax.axis_index("s") * NC + lax.axis_index("c")
        count = base_cnt + jnp.minimum(jnp.maximum(extra - wid, 0), 1)
        f_lo = lax.iota(jnp.int32, L)
        fvecs = [
            f0 + jnp.bitwise_and(d + f_lo, L - 1)
            for f0 in (0, L)
            for d in range(L)
        ]
        stages = (st0, st1)
        obufs = (ob0, ob1)
        lsems = (ls0, ls1)
        wsems = (ws0, ws1)

        for tabT, tl, out in ((utabT, utail, uout), (itabT, itail, iout)):
            def load(i, slot, tabT=tabT):
                @pl.when(i < count)
                def _():
                    sc = wid + i * NW
                    pltpu.async_copy(
                        tabT.at[:, pl.ds(sc * W, W)], stages[slot], lsems[slot]
                    )

            def proc(i, slot, tabT=tabT, out=out):
                @pl.when(i < count)
                def _():
                    sc = wid + i * NW
                    stage, obuf = stages[slot], obufs[slot]
                    pltpu.make_async_copy(
                        tabT.at[:, pl.ds(0, W)], stage, lsems[slot]
                    ).wait()

                    @functools.partial(plsc.parallel_loop, 0, W // L, unroll=4)
                    def _per_blk(b):
                        cvec = b * L + f_lo
                        pos_base = cvec * D
                        for fv in fvecs:
                            v = plsc.load_gather(stage, [fv, cvec])
                            plsc.store_scatter(obuf, [pos_base + fv], v)
                    @pl.when(i >= 2)
                    def _w():
                        pltpu.make_async_copy(
                            out.at[pl.ds(0, W * D)], obuf, wsems[slot]
                        ).wait()

                    pltpu.async_copy(
                        obuf, out.at[pl.ds(sc * W * D, W * D)], wsems[slot]
                    )

            load(0, 0)

            def pair(g, _):
                i0, i1 = 2 * g, 2 * g + 1
                load(i1, 1)
                proc(i0, 0)
                load(i1 + 1, 0)
                proc(i1, 1)
                return _

            lax.fori_loop(0, (base_cnt + 2) // 2, pair, 0)
            for slot in (0, 1):
                @pl.when(count > slot)
                def _fd(slot=slot, out=out):
                    pltpu.make_async_copy(
                        out.at[pl.ds(0, W * D)], obufs[slot], wsems[slot]
                    ).wait()

            @pl.when(wid == 0)
            def _copy_tail(tl=tl, out=out):
                pltpu.sync_copy(tl, out.at[pl.ds(n_full * TILE * D, tail * D)])

    return detile_k


@functools.lru_cache(maxsize=None)
def _make_gather(B, D):
    info = plsc.get_sparse_core_info()
    NC, NS = info.num_cores, info.num_subcores
    NW = NC * NS
    b_per_w = B // NW
    n_chunks = b_per_w // CHUNK
    mesh = plsc.VectorSubcoreMesh(core_axis_name="c", subcore_axis_name="s")

    @functools.partial(
        pl.kernel,
        mesh=mesh,
        out_type=[
            jax.ShapeDtypeStruct((B, D), jnp.float32),
            jax.ShapeDtypeStruct((B, D), jnp.float32),
        ],
        scratch_types=[
            pltpu.VMEM((n_chunks, CHUNK), jnp.int32),
            pltpu.VMEM((n_chunks, CHUNK), jnp.int32),
            pltpu.VMEM((b_per_w, D), jnp.float32),
            pltpu.VMEM((b_per_w, D), jnp.float32),
            pltpu.SemaphoreType.DMA,
        ],
        compiler_params=pltpu.CompilerParams(use_tc_tiling_on_sc=False),
    )
    def gather_k(utab, itab, uidx, iidx, uout, iout, uidx_v, iidx_v, urows, irows, sem):
        wid = lax.axis_index("s") * NC + lax.axis_index("c")
        row0 = wid * n_chunks
        pltpu.sync_copy(uidx.at[pl.ds(row0, n_chunks)], uidx_v)
        pltpu.sync_copy(iidx.at[pl.ds(row0, n_chunks)], iidx_v)
        copies = []
        for j in range(n_chunks):
            copies.append(
                pltpu.async_copy(utab.at[uidx_v.at[j]], urows.at[pl.ds(j * CHUNK, CHUNK)], sem)
            )
            copies.append(
                pltpu.async_copy(itab.at[iidx_v.at[j]], irows.at[pl.ds(j * CHUNK, CHUNK)], sem)
            )
        for c in copies:
            c.wait()
        base = wid * b_per_w
        pltpu.sync_copy(urows, uout.at[pl.ds(base, b_per_w)])
        pltpu.sync_copy(irows, iout.at[pl.ds(base, b_per_w)])

    return gather_k


def _desc_body(desc, Wd, bd, out):
    out[...] = jnp.maximum(
        jnp.dot(desc[...], Wd[...], preferred_element_type=jnp.float32) + bd[...], 0.0
    )


def _desc_proj(desc, Wd, bd):
    B, K = desc.shape
    D = Wd.shape[1]
    BB = 2048
    return pl.pallas_call(
        _desc_body,
        grid=(B // BB,),
        in_specs=[
            pl.BlockSpec((BB, K), lambda i: (i, 0)),
            pl.BlockSpec(Wd.shape, lambda i: (0, 0)),
            pl.BlockSpec(bd.shape, lambda i: (0, 0)),
        ],
        out_specs=pl.BlockSpec((BB, D), lambda i: (i, 0)),
        out_shape=jax.ShapeDtypeStruct((B, D), jnp.float32),
    )(desc, Wd, bd)


def _mlp_body(uemb, iemb, dd, W1, b1, W2, b2, Wo, bo, out):
    W1v = W1[...]
    h = (
        jnp.dot(uemb[...], W1v[0:EMBED], preferred_element_type=jnp.float32)
        + jnp.dot(iemb[...], W1v[EMBED : 2 * EMBED], preferred_element_type=jnp.float32)
        + jnp.dot(dd[...], W1v[2 * EMBED :], preferred_element_type=jnp.float32)
        + b1[...]
    )
    h = jnp.maximum(h, 0.0)
    h2 = jnp.maximum(
        jnp.dot(h, W2[...], preferred_element_type=jnp.float32) + b2[...], 0.0
    )
    out[...] = jnp.sum(h2 * Wo[...], axis=1, keepdims=True) + bo[...]


def _mlp(uemb, iemb, dd, W1, b1, W2, b2, Wo, bo):
    B, D = uemb.shape
    BB = 4096

    def row_blk(shape):
        return pl.BlockSpec(shape, lambda i: (i, 0))

    def full_blk(shape):
        return pl.BlockSpec(shape, lambda i: (0, 0))

    return pl.pallas_call(
        _mlp_body,
        grid=(B // BB,),
        in_specs=[
            row_blk((BB, D)),
            row_blk((BB, D)),
            row_blk((BB, D)),
            full_blk(W1.shape),
            full_blk(b1.shape),
            full_blk(W2.shape),
            full_blk(b2.shape),
            full_blk(Wo.shape),
            full_blk(bo.shape),
        ],
        out_specs=row_blk((BB, 1)),
        out_shape=jax.ShapeDtypeStruct((B, 1), jnp.float32),
    )(uemb, iemb, dd, W1, b1, W2, b2, Wo, bo)


@jax.jit
def kernel(user_input, item_input, description_input, user_table, item_table,
           W_desc, b_desc, W1, b1, W2, b2, W_out, b_out):
    B = user_input.shape[0]
    N = user_table.shape[0]
    uidx = user_input.reshape(B // CHUNK, CHUNK)
    iidx = item_input.reshape(B // CHUNK, CHUNK)
    split = (N // TILE) * TILE
    ulin, ilin = _make_detile(N, EMBED)(
        user_table.T, item_table.T,
        user_table[split:].reshape(-1), item_table[split:].reshape(-1),
    )
    uemb, iemb = _make_gather(B, EMBED)(
        ulin.reshape(N, EMBED), ilin.reshape(N, EMBED), uidx, iidx
    )
    dd = _desc_proj(description_input, W_desc, b_desc.reshape(1, -1))
    return _mlp(
        uemb, iemb, dd,
        W1, b1.reshape(1, -1),
        W2, b2.reshape(1, -1),
        W_out.reshape(1, -1), b_out.reshape(1, -1),
    )
